# Initial kernel scaffold; baseline (speedup 1.0000x reference)
import jax
import jax.numpy as jnp
from jax.experimental import pallas as pl


def kernel(x, batch, W1, b1, W2, b2):
    raise NotImplementedError("write your pallas kernel here")


# speedup vs baseline: 2.7348x; 2.7348x over previous
import functools

import jax
import jax.numpy as jnp
from jax.experimental import pallas as pl
from jax.experimental.pallas import tpu as pltpu


def _pick_tile(n: int) -> int:
    best = 0
    for d in range(8, 1025, 8):
        if n % d == 0:
            best = d
    if best == 0:
        raise ValueError(f"no tile size divides {n}")
    return best


def _build_items(batch, n, tile_rows, num_tiles, g):
    gid = jnp.arange(g, dtype=jnp.int32)
    starts = jnp.searchsorted(batch, gid, side="left").astype(jnp.int32)
    ends = jnp.searchsorted(batch, gid, side="right").astype(jnp.int32)
    nonempty = ends > starts
    t0 = starts // tile_rows
    t1 = jnp.where(nonempty, (ends - 1) // tile_rows, t0)
    counts = jnp.where(nonempty, t1 - t0 + 1, 1).astype(jnp.int32)
    offsets = jnp.concatenate(
        [jnp.zeros((1,), jnp.int32), jnp.cumsum(counts).astype(jnp.int32)]
    )
    total = offsets[-1]
    num_items = num_tiles + g - 1
    i = jnp.arange(num_items, dtype=jnp.int32)
    seg = jnp.clip(
        jnp.searchsorted(offsets, i, side="right").astype(jnp.int32) - 1, 0, g - 1
    )
    k = i - offsets[seg]
    tile = jnp.clip(t0[seg] + k, 0, num_tiles - 1)
    lo = jnp.clip(starts[seg] - tile * tile_rows, 0, tile_rows)
    hi = jnp.clip(ends[seg] - tile * tile_rows, 0, tile_rows)
    valid = i < total
    lo = jnp.where(valid, lo, 0)
    hi = jnp.maximum(jnp.where(valid, hi, 0), lo)
    is_last = (valid & (k == counts[seg] - 1)).astype(jnp.int32)
    return seg, tile, lo, hi, is_last


def _pool_body(tile_rows, seg_sp, tile_sp, lo_sp, hi_sp, last_sp,
               x_ref, w1_ref, b1_ref, w2_ref, b2_ref, out_ref, e_ref, den_ref):
    i = pl.program_id(0)
    prev = jnp.maximum(i - 1, 0)
    tile_new = jnp.logical_or(i == 0, tile_sp[i] != tile_sp[prev])
    seg_new = jnp.logical_or(i == 0, seg_sp[i] != seg_sp[prev])

    @pl.when(tile_new)
    def _():
        xb = x_ref[...]
        h = jnp.tanh(
            jnp.dot(xb, w1_ref[...], preferred_element_type=jnp.float32)
            + b1_ref[...]
        )
        s = jnp.dot(h, w2_ref[...], preferred_element_type=jnp.float32)
        bound = jnp.sum(jnp.abs(w2_ref[...])) + jnp.abs(b2_ref[0, 0])
        e_ref[...] = jnp.exp(s + (b2_ref[0, 0] - bound))

    lo = lo_sp[i]
    hi = hi_sp[i]
    rows = jax.lax.broadcasted_iota(jnp.int32, (tile_rows, 1), 0)
    ew = jnp.where((rows >= lo) & (rows < hi), e_ref[...], 0.0)
    num = jax.lax.dot_general(
        ew, x_ref[...], (((0,), (0,)), ((), ())),
        preferred_element_type=jnp.float32,
    )
    dsum = jnp.sum(ew, axis=(0, 1), keepdims=True)

    @pl.when(seg_new)
    def _():
        out_ref[0] = num
        den_ref[...] = dsum

    @pl.when(jnp.logical_not(seg_new))
    def _():
        out_ref[0] += num
        den_ref[...] += dsum

    @pl.when(last_sp[i] != 0)
    def _():
        d = den_ref[...]
        out_ref[0] = out_ref[0] / jnp.where(d == 0.0, 1.0, d)


def kernel(x, batch, W1, b1, W2, b2):
    n, d = x.shape
    g = 512
    dh = W1.shape[1]
    tile_rows = _pick_tile(n)
    num_tiles = n // tile_rows
    batch = batch.astype(jnp.int32)
    seg, tile, lo, hi, is_last = _build_items(batch, n, tile_rows, num_tiles, g)

    grid_spec = pltpu.PrefetchScalarGridSpec(
        num_scalar_prefetch=5,
        grid=(num_tiles + g - 1,),
        in_specs=[
            pl.BlockSpec((tile_rows, d), lambda i, seg, tile, lo, hi, last: (tile[i], 0)),
            pl.BlockSpec((d, dh), lambda i, *_: (0, 0)),
            pl.BlockSpec((1, dh), lambda i, *_: (0, 0)),
            pl.BlockSpec((dh, 1), lambda i, *_: (0, 0)),
            pl.BlockSpec((1, 1), lambda i, *_: (0, 0)),
        ],
        out_specs=pl.BlockSpec((1, 1, d), lambda i, seg, *_: (seg[i], 0, 0)),
        scratch_shapes=[
            pltpu.VMEM((tile_rows, 1), jnp.float32),
            pltpu.VMEM((1, 1), jnp.float32),
        ],
    )
    out = pl.pallas_call(
        functools.partial(_pool_body, tile_rows),
        grid_spec=grid_spec,
        out_shape=jax.ShapeDtypeStruct((g, 1, d), jnp.float32),
    )(seg, tile, lo, hi, is_last,
      x, W1, b1.reshape(1, dh), W2, b2.reshape(1, 1))
    return out.reshape(g, d)


# speedup vs baseline: 9.1406x; 3.3424x over previous
import functools

import jax
import jax.numpy as jnp
from jax import lax
from jax.experimental import pallas as pl
from jax.experimental.pallas import tpu as pltpu

_G = 512
_W = 64
_OUTW = 64


def _pick_tile(n: int) -> int:
    best = 0
    for d in range(8, 1025, 8):
        if n % d == 0:
            best = d
    if best == 0:
        raise ValueError(f"no tile size divides {n}")
    return best


def _body(tile_rows, num_tiles, tfirst_sp, nwin_sp,
          x_ref, bcol_ref, w1_ref, b1_ref, w2_ref, b2_ref,
          out_ref, acc_ref, den_ref):
    i = pl.program_id(0)

    @pl.when(i == 0)
    def _():
        acc_ref[...] = jnp.zeros_like(acc_ref)
        den_ref[...] = jnp.zeros_like(den_ref)

    @pl.when(i < num_tiles)
    def _():
        xb = x_ref[...]
        h = jnp.tanh(
            jnp.dot(xb, w1_ref[...], preferred_element_type=jnp.float32)
            + b1_ref[...]
        )
        s = jnp.dot(h, w2_ref[...], preferred_element_type=jnp.float32)
        bound = jnp.sum(jnp.abs(w2_ref[...])) + jnp.abs(b2_ref[0, 0])
        e = jnp.exp(s + (b2_ref[0, 0] - bound))
        ex = xb * e
        g0 = tfirst_sp[i]
        rel = bcol_ref[...] - g0.astype(jnp.float32)
        cw = lax.broadcasted_iota(jnp.int32, (tile_rows, _W), 1).astype(jnp.float32)

        def win(j, carry):
            colid = rel - (j * _W).astype(jnp.float32)
            onehot = jnp.where(colid == cw, 1.0, 0.0)
            numw = lax.dot_general(
                onehot, ex, (((0,), (0,)), ((), ())),
                preferred_element_type=jnp.float32,
            )
            denw = lax.dot_general(
                onehot, e, (((0,), (0,)), ((), ())),
                preferred_element_type=jnp.float32,
            )
            base = g0 + j * _W
            acc_ref[pl.ds(base, _W), :] += numw
            den_ref[pl.ds(base, _W), :] += denw
            return carry

        lax.fori_loop(0, nwin_sp[i], win, 0)

    @pl.when(i >= num_tiles)
    def _():
        b = i - num_tiles
        a = acc_ref[pl.ds(b * _OUTW, _OUTW), :]
        dd = den_ref[pl.ds(b * _OUTW, _OUTW), :]
        out_ref[...] = a / jnp.where(dd == 0.0, 1.0, dd)


def kernel(x, batch, W1, b1, W2, b2):
    n, d = x.shape
    dh = W1.shape[1]
    tile_rows = _pick_tile(n)
    num_tiles = n // tile_rows
    out_steps = _G // _OUTW
    grid = num_tiles + out_steps

    batch = batch.astype(jnp.int32)
    bcol = batch.astype(jnp.float32).reshape(n, 1)
    tidx = jnp.arange(num_tiles, dtype=jnp.int32) * tile_rows
    tfirst = batch[tidx]
    tlast = batch[tidx + tile_rows - 1]
    nwin = (tlast - tfirst) // _W + 1
    pad = jnp.zeros((out_steps,), jnp.int32)
    tfirst = jnp.concatenate([tfirst, pad])
    nwin = jnp.concatenate([nwin, pad])

    grid_spec = pltpu.PrefetchScalarGridSpec(
        num_scalar_prefetch=2,
        grid=(grid,),
        in_specs=[
            pl.BlockSpec((tile_rows, d),
                         lambda i, tf, nw: (jnp.minimum(i, num_tiles - 1), 0)),
            pl.BlockSpec((tile_rows, 1),
                         lambda i, tf, nw: (jnp.minimum(i, num_tiles - 1), 0)),
            pl.BlockSpec((d, dh), lambda i, *_: (0, 0)),
            pl.BlockSpec((1, dh), lambda i, *_: (0, 0)),
            pl.BlockSpec((dh, 1), lambda i, *_: (0, 0)),
            pl.BlockSpec((1, 1), lambda i, *_: (0, 0)),
        ],
        out_specs=pl.BlockSpec(
            (_OUTW, d), lambda i, *_: (jnp.maximum(i - num_tiles, 0), 0)),
        scratch_shapes=[
            pltpu.VMEM((_G + _W, d), jnp.float32),
            pltpu.VMEM((_G + _W, 1), jnp.float32),
        ],
    )
    out = pl.pallas_call(
        functools.partial(_body, tile_rows, num_tiles),
        grid_spec=grid_spec,
        out_shape=jax.ShapeDtypeStruct((_G, d), jnp.float32),
    )(tfirst, nwin,
      x, bcol, W1, b1.reshape(1, dh), W2, b2.reshape(1, 1))
    return out


# speedup vs baseline: 12.3128x; 1.3470x over previous
import functools

import jax
import jax.numpy as jnp
from jax import lax
from jax.experimental import pallas as pl
from jax.experimental.pallas import tpu as pltpu

_G = 512
_W = 16
_OUTW = 64


def _pick_tile(n: int) -> int:
    best = 0
    for d in range(8, 2049, 8):
        if n % d == 0:
            best = d
    if best == 0:
        raise ValueError(f"no tile size divides {n}")
    return best


def _body(tile_rows, num_tiles, tfirst_sp, nwin_sp,
          x_ref, bcol_ref, w1_ref, b1_ref, w2_ref, b2_ref,
          out_ref, acc_ref, den_ref):
    i = pl.program_id(0)

    @pl.when(i == 0)
    def _():
        acc_ref[...] = jnp.zeros_like(acc_ref)
        den_ref[...] = jnp.zeros_like(den_ref)

    @pl.when(i < num_tiles)
    def _():
        xb = x_ref[...]
        h = jnp.tanh(
            jnp.dot(xb, w1_ref[...], preferred_element_type=jnp.float32)
            + b1_ref[...]
        )
        s = jnp.dot(h, w2_ref[...], preferred_element_type=jnp.float32)
        bound = jnp.sum(jnp.abs(w2_ref[...])) + jnp.abs(b2_ref[0, 0])
        shift = b2_ref[0, 0] - bound
        eb = jnp.exp(jnp.broadcast_to(s, (tile_rows, _W)) + shift)
        bcolb = bcol_ref[...]
        cw = lax.broadcasted_iota(jnp.int32, (tile_rows, _W), 1).astype(jnp.float32)
        ones_col = jnp.ones((tile_rows, 1), jnp.float32)
        g0 = tfirst_sp[i]

        def win(j, carry):
            off = (g0 + j * _W).astype(jnp.float32)
            oe = jnp.where(bcolb == cw + off, eb, 0.0)
            numw = lax.dot_general(
                oe, xb, (((0,), (0,)), ((), ())),
                preferred_element_type=jnp.float32,
            )
            denw = lax.dot_general(
                oe, ones_col, (((0,), (0,)), ((), ())),
                preferred_element_type=jnp.float32,
            )
            base = g0 + j * _W
            acc_ref[pl.ds(base, _W), :] += numw
            den_ref[pl.ds(base, _W), :] += denw
            return carry

        lax.fori_loop(0, nwin_sp[i], win, 0)

    @pl.when(i >= num_tiles)
    def _():
        b = i - num_tiles
        a = acc_ref[pl.ds(b * _OUTW, _OUTW), :]
        dd = den_ref[pl.ds(b * _OUTW, _OUTW), :]
        out_ref[...] = a / jnp.where(dd == 0.0, 1.0, dd)


def kernel(x, batch, W1, b1, W2, b2):
    n, d = x.shape
    dh = W1.shape[1]
    tile_rows = _pick_tile(n)
    num_tiles = n // tile_rows
    out_steps = _G // _OUTW
    grid = num_tiles + out_steps

    batch = batch.astype(jnp.int32)
    bcol = batch.astype(jnp.float32).reshape(n, 1)
    tidx = jnp.arange(num_tiles, dtype=jnp.int32) * tile_rows
    tfirst = batch[tidx]
    tlast = batch[tidx + tile_rows - 1]
    nwin = (tlast - tfirst) // _W + 1
    pad = jnp.zeros((out_steps,), jnp.int32)
    tfirst = jnp.concatenate([tfirst, pad])
    nwin = jnp.concatenate([nwin, pad])

    grid_spec = pltpu.PrefetchScalarGridSpec(
        num_scalar_prefetch=2,
        grid=(grid,),
        in_specs=[
            pl.BlockSpec((tile_rows, d),
                         lambda i, tf, nw: (jnp.minimum(i, num_tiles - 1), 0)),
            pl.BlockSpec((tile_rows, 1),
                         lambda i, tf, nw: (jnp.minimum(i, num_tiles - 1), 0)),
            pl.BlockSpec((d, dh), lambda i, *_: (0, 0)),
            pl.BlockSpec((1, dh), lambda i, *_: (0, 0)),
            pl.BlockSpec((dh, 1), lambda i, *_: (0, 0)),
            pl.BlockSpec((1, 1), lambda i, *_: (0, 0)),
        ],
        out_specs=pl.BlockSpec(
            (_OUTW, d), lambda i, *_: (jnp.maximum(i - num_tiles, 0), 0)),
        scratch_shapes=[
            pltpu.VMEM((_G + _W, d), jnp.float32),
            pltpu.VMEM((_G + _W, 1), jnp.float32),
        ],
    )
    out = pl.pallas_call(
        functools.partial(_body, tile_rows, num_tiles),
        grid_spec=grid_spec,
        out_shape=jax.ShapeDtypeStruct((_G, d), jnp.float32),
    )(tfirst, nwin,
      x, bcol, W1, b1.reshape(1, dh), W2, b2.reshape(1, 1))
    return out


# speedup vs baseline: 24.4123x; 1.9827x over previous
import functools

import jax
import jax.numpy as jnp
from jax import lax
from jax.experimental import pallas as pl
from jax.experimental.pallas import tpu as pltpu

_G = 512
_W = 16
_OUTW = 64


def _pick_tile(n: int) -> int:
    best = 0
    for d in range(8, 2049, 8):
        if n % d == 0:
            best = d
    if best == 0:
        raise ValueError(f"no tile size divides {n}")
    return best


def _body(tile_rows, num_tiles, tfirst_sp, nwin_sp,
          x_ref, brow_ref, w1_ref, b1_ref, w2_ref, b2_ref,
          out_ref, acc_ref, den_ref):
    i = pl.program_id(0)

    @pl.when(i == 0)
    def _():
        acc_ref[...] = jnp.zeros_like(acc_ref)
        den_ref[...] = jnp.zeros_like(den_ref)

    @pl.when(i < num_tiles)
    def _():
        xb = x_ref[...]
        ones_row = jnp.ones((1, tile_rows), jnp.float32)
        ht = lax.dot_general(
            w1_ref[...], xb, (((0,), (1,)), ((), ())),
            preferred_element_type=jnp.float32,
        )
        bb = lax.dot_general(
            b1_ref[...], ones_row, (((0,), (0,)), ((), ())),
            preferred_element_type=jnp.float32,
        )
        ht = jnp.tanh(ht + bb)
        st = lax.dot_general(
            w2_ref[...], ht, (((0,), (0,)), ((), ())),
            preferred_element_type=jnp.float32,
        )
        bound = jnp.sum(jnp.abs(w2_ref[...])) + jnp.abs(b2_ref[0, 0])
        et = jnp.exp(st + (b2_ref[0, 0] - bound))
        bt = brow_ref[0]
        etb = jnp.broadcast_to(et, (_W, tile_rows))
        btb = jnp.broadcast_to(bt, (_W, tile_rows))
        cw = lax.broadcasted_iota(jnp.int32, (_W, tile_rows), 0).astype(jnp.float32)
        g0 = tfirst_sp[i]

        def win(j, carry):
            off = (g0 + j * _W).astype(jnp.float32)
            oet = jnp.where(btb == cw + off, etb, 0.0)
            numw = lax.dot_general(
                oet, xb, (((1,), (0,)), ((), ())),
                preferred_element_type=jnp.float32,
            )
            denw = jnp.sum(oet, axis=1, keepdims=True)
            base = g0 + j * _W
            acc_ref[pl.ds(base, _W), :] += numw
            den_ref[pl.ds(base, _W), :] += denw
            return carry

        lax.fori_loop(0, nwin_sp[i], win, 0)

    @pl.when(i >= num_tiles)
    def _():
        b = i - num_tiles
        a = acc_ref[pl.ds(b * _OUTW, _OUTW), :]
        dd = den_ref[pl.ds(b * _OUTW, _OUTW), :]
        out_ref[...] = a / jnp.where(dd == 0.0, 1.0, dd)


def kernel(x, batch, W1, b1, W2, b2):
    n, d = x.shape
    dh = W1.shape[1]
    tile_rows = _pick_tile(n)
    num_tiles = n // tile_rows
    out_steps = _G // _OUTW
    grid = num_tiles + out_steps

    batch = batch.astype(jnp.int32)
    brow = batch.astype(jnp.float32).reshape(num_tiles, 1, tile_rows)
    tidx = jnp.arange(num_tiles, dtype=jnp.int32) * tile_rows
    tfirst = batch[tidx]
    tlast = batch[tidx + tile_rows - 1]
    nwin = (tlast - tfirst) // _W + 1
    pad = jnp.zeros((out_steps,), jnp.int32)
    tfirst = jnp.concatenate([tfirst, pad])
    nwin = jnp.concatenate([nwin, pad])

    grid_spec = pltpu.PrefetchScalarGridSpec(
        num_scalar_prefetch=2,
        grid=(grid,),
        in_specs=[
            pl.BlockSpec((tile_rows, d),
                         lambda i, tf, nw: (jnp.minimum(i, num_tiles - 1), 0)),
            pl.BlockSpec((1, 1, tile_rows),
                         lambda i, tf, nw: (jnp.minimum(i, num_tiles - 1), 0, 0)),
            pl.BlockSpec((d, dh), lambda i, *_: (0, 0)),
            pl.BlockSpec((1, dh), lambda i, *_: (0, 0)),
            pl.BlockSpec((dh, 1), lambda i, *_: (0, 0)),
            pl.BlockSpec((1, 1), lambda i, *_: (0, 0)),
        ],
        out_specs=pl.BlockSpec(
            (_OUTW, d), lambda i, *_: (jnp.maximum(i - num_tiles, 0), 0)),
        scratch_shapes=[
            pltpu.VMEM((_G + _W, d), jnp.float32),
            pltpu.VMEM((_G + _W, 1), jnp.float32),
        ],
    )
    out = pl.pallas_call(
        functools.partial(_body, tile_rows, num_tiles),
        grid_spec=grid_spec,
        out_shape=jax.ShapeDtypeStruct((_G, d), jnp.float32),
    )(tfirst, nwin,
      x, brow, W1, b1.reshape(1, dh), W2, b2.reshape(1, 1))
    return out


# speedup vs baseline: 31.5068x; 1.2906x over previous
import functools

import jax
import jax.numpy as jnp
from jax import lax
from jax.experimental import pallas as pl
from jax.experimental.pallas import tpu as pltpu

_G = 512
_W = 16
_OUTW = 64


def _pick_tile(n: int) -> int:
    best = 0
    for d in range(8, 4097, 8):
        if n % d == 0:
            best = d
    if best == 0:
        raise ValueError(f"no tile size divides {n}")
    return best


def _body(tile_rows, num_tiles, tfirst_sp, nwin_sp,
          x_ref, brow_ref, w1_ref, b1_ref, w2_ref, b2_ref,
          out_ref, acc_ref, den_ref):
    i = pl.program_id(0)

    @pl.when(i == 0)
    def _():
        acc_ref[...] = jnp.zeros_like(acc_ref)
        den_ref[...] = jnp.zeros_like(den_ref)

    @pl.when(i < num_tiles)
    def _():
        xb = x_ref[...]
        ones_row = jnp.ones((1, tile_rows), jnp.float32)
        ht = lax.dot_general(
            w1_ref[...], xb, (((0,), (1,)), ((), ())),
            preferred_element_type=jnp.float32,
        )
        bb = lax.dot_general(
            b1_ref[...], ones_row, (((0,), (0,)), ((), ())),
            preferred_element_type=jnp.float32,
        )
        ht = jnp.tanh(ht + bb)
        st = lax.dot_general(
            w2_ref[...], ht, (((0,), (0,)), ((), ())),
            preferred_element_type=jnp.float32,
        )
        bound = jnp.sum(jnp.abs(w2_ref[...])) + jnp.abs(b2_ref[0, 0])
        et = jnp.exp(st + (b2_ref[0, 0] - bound))
        bt = brow_ref[0]
        etb = jnp.broadcast_to(et, (_W, tile_rows))
        btb = jnp.broadcast_to(bt, (_W, tile_rows))
        cw = lax.broadcasted_iota(jnp.int32, (_W, tile_rows), 0).astype(jnp.float32)
        g0 = tfirst_sp[i]

        def win(j, carry):
            off = (g0 + j * _W).astype(jnp.float32)
            oet = jnp.where(btb == cw + off, etb, 0.0)
            numw = lax.dot_general(
                oet, xb, (((1,), (0,)), ((), ())),
                preferred_element_type=jnp.float32,
            )
            denw = jnp.sum(oet, axis=1, keepdims=True)
            base = g0 + j * _W
            acc_ref[pl.ds(base, _W), :] += numw
            den_ref[pl.ds(base, _W), :] += denw
            return carry

        lax.fori_loop(0, nwin_sp[i], win, 0)

    @pl.when(i >= num_tiles)
    def _():
        b = i - num_tiles
        a = acc_ref[pl.ds(b * _OUTW, _OUTW), :]
        dd = den_ref[pl.ds(b * _OUTW, _OUTW), :]
        out_ref[...] = a / jnp.where(dd == 0.0, 1.0, dd)


def kernel(x, batch, W1, b1, W2, b2):
    n, d = x.shape
    dh = W1.shape[1]
    tile_rows = _pick_tile(n)
    num_tiles = n // tile_rows
    out_steps = _G // _OUTW
    grid = num_tiles + out_steps

    batch = batch.astype(jnp.int32)
    brow = batch.astype(jnp.float32).reshape(num_tiles, 1, tile_rows)
    tidx = jnp.arange(num_tiles, dtype=jnp.int32) * tile_rows
    tfirst = batch[tidx]
    tlast = batch[tidx + tile_rows - 1]
    nwin = (tlast - tfirst) // _W + 1
    pad = jnp.zeros((out_steps,), jnp.int32)
    tfirst = jnp.concatenate([tfirst, pad])
    nwin = jnp.concatenate([nwin, pad])

    grid_spec = pltpu.PrefetchScalarGridSpec(
        num_scalar_prefetch=2,
        grid=(grid,),
        in_specs=[
            pl.BlockSpec((tile_rows, d),
                         lambda i, tf, nw: (jnp.minimum(i, num_tiles - 1), 0)),
            pl.BlockSpec((1, 1, tile_rows),
                         lambda i, tf, nw: (jnp.minimum(i, num_tiles - 1), 0, 0)),
            pl.BlockSpec((d, dh), lambda i, *_: (0, 0)),
            pl.BlockSpec((1, dh), lambda i, *_: (0, 0)),
            pl.BlockSpec((dh, 1), lambda i, *_: (0, 0)),
            pl.BlockSpec((1, 1), lambda i, *_: (0, 0)),
        ],
        out_specs=pl.BlockSpec(
            (_OUTW, d), lambda i, *_: (jnp.maximum(i - num_tiles, 0), 0)),
        scratch_shapes=[
            pltpu.VMEM((_G + _W, d), jnp.float32),
            pltpu.VMEM((_G + _W, 1), jnp.float32),
        ],
    )
    out = pl.pallas_call(
        functools.partial(_body, tile_rows, num_tiles),
        grid_spec=grid_spec,
        out_shape=jax.ShapeDtypeStruct((_G, d), jnp.float32),
    )(tfirst, nwin,
      x, brow, W1, b1.reshape(1, dh), W2, b2.reshape(1, 1))
    return out
